# baseline (speedup 1.0000x reference)
import functools

import jax
import jax.numpy as jnp
from jax import lax
from jax.experimental import pallas as pl
from jax.experimental.pallas import tpu as pltpu
from jax.experimental.pallas import tpu_sc as plsc

NUM_ROWS = 1000000
DIM = 64
BATCH = 4096
SEQ = 200

NC = 2
NS = 16
NW = NC * NS
L = 16

VROWS = NUM_ROWS // 2
N_SLABS_FULL = NUM_ROWS // 128
TAIL_M = N_SLABS_FULL
K1 = (N_SLABS_FULL + NW - 1) // NW + 1
BBLK = BATCH // NW

_mesh = plsc.VectorSubcoreMesh(core_axis_name="c", subcore_axis_name="s")
_params = pltpu.CompilerParams(use_tc_tiling_on_sc=True, needs_layout_passes=False)


def _wid():
    return lax.axis_index("s") * NC + lax.axis_index("c")


def _iota16():
    return lax.broadcasted_iota(jnp.int32, (L,), 0)


@functools.partial(
    pl.kernel,
    out_type=jax.ShapeDtypeStruct((VROWS, 128), jnp.float32),
    mesh=_mesh,
    scratch_types=[
        pltpu.VMEM((2, 64, 128), jnp.float32),
        pltpu.VMEM((2, 64, 128), jnp.float32),
        pltpu.SemaphoreType.DMA,
        pltpu.SemaphoreType.DMA,
        pltpu.SemaphoreType.DMA,
        pltpu.SemaphoreType.DMA,
    ],
    compiler_params=_params,
)
def _format_kernel(nt_hbm, tail_hbm, out_hbm, tin, tout, gi0, gi1, go0, go1):
    w = _wid()
    iota = _iota16()
    gsem = (gi0, gi1)
    osem = (go0, go1)

    def m_of(k):
        return w + k * NW

    def fire_in(k, b):
        col0 = pl.multiple_of(m_of(k) * 128, 128)
        pltpu.async_copy(nt_hbm.at[:, pl.ds(col0, 128)], tin.at[b], gsem[b])

    def wait_in(b):
        pltpu.make_async_copy(nt_hbm.at[:, pl.ds(0, 128)], tin.at[b], gsem[b]).wait()

    def transpose_slab(b, n_u):
        src = tin.at[b]
        dst = tout.at[b]

        @pl.loop(0, n_u, step=4)
        def _ul(u0):
            for du in range(4):
                u = u0 + du
                for half in range(2):
                    col = jnp.broadcast_to((2 * u + half).astype(jnp.int32), (L,))
                    for dbase in range(0, DIM, L):
                        vals = plsc.load_gather(src, [iota + dbase, col])
                        dst[u, pl.ds(half * DIM + dbase, L)] = vals

    def fire_out(k, b):
        row0 = pl.multiple_of(m_of(k) * 64, 64)
        pltpu.async_copy(tout.at[b], out_hbm.at[pl.ds(row0, 64)], osem[b])

    def wait_out(b):
        pltpu.make_async_copy(tout.at[b], out_hbm.at[pl.ds(0, 64)], osem[b]).wait()

    @pl.when(m_of(0) < N_SLABS_FULL)
    def _():
        fire_in(0, 0)

    @pl.when(m_of(1) < N_SLABS_FULL)
    def _():
        fire_in(1, 1)

    @pl.loop(0, K1 + 1, step=2)
    def _step(c):
        for b in range(2):
            k = c + b

            @pl.when(m_of(k) < N_SLABS_FULL)
            def _proc():
                wait_in(b)

                @pl.when(k >= 2)
                def _():
                    wait_out(b)

                transpose_slab(b, 64)
                fire_out(k, b)

                @pl.when(m_of(k + 2) < N_SLABS_FULL)
                def _():
                    fire_in(k + 2, b)

    @pl.when(m_of(0) < N_SLABS_FULL)
    def _():
        wait_out(0)

    @pl.when(m_of(1) < N_SLABS_FULL)
    def _():
        wait_out(1)

    @pl.when(w == TAIL_M % NW)
    def _tail():
        pltpu.sync_copy(tail_hbm, tin.at[0, pl.ds(0, 32)])
        pltpu.sync_copy(tin.at[0, pl.ds(0, 32)], out_hbm.at[pl.ds(TAIL_M * 64, 32)])


@functools.partial(
    pl.kernel,
    out_type=jax.ShapeDtypeStruct((SEQ, DIM, BATCH), jnp.float32),
    mesh=_mesh,
    scratch_types=[
        pltpu.VMEM((SEQ, BBLK), jnp.int32),
        pltpu.VMEM((2, BBLK), jnp.int32),
        pltpu.VMEM((2, BBLK, 128), jnp.float32),
        pltpu.VMEM((2, DIM, BBLK), jnp.float32),
        pltpu.SemaphoreType.DMA,
        pltpu.SemaphoreType.DMA,
        pltpu.SemaphoreType.DMA,
        pltpu.SemaphoreType.DMA,
        pltpu.SemaphoreType.DMA,
    ],
    compiler_params=_params,
)
def _gather_kernel(
    ids_hbm, tab_hbm, out_hbm, ids_v, vlist, gbuf, obuf, gs0, gs1, os0, os1, isem
):
    w = _wid()
    iota = _iota16()
    b0 = pl.multiple_of(w * BBLK, BBLK)
    gsem = (gs0, gs1)
    osem = (os0, os1)

    pltpu.async_copy(ids_hbm.at[:, pl.ds(b0, BBLK)], ids_v, isem)
    pltpu.make_async_copy(ids_hbm.at[:, pl.ds(0, BBLK)], ids_v, isem).wait()

    def prep(s, b):
        for g in range(BBLK // L):
            ids = ids_v[s, pl.ds(g * L, L)]
            vlist[b, pl.ds(g * L, L)] = lax.shift_right_logical(ids, 1)

    def fire_gather(s, b):
        pltpu.async_copy(tab_hbm.at[vlist.at[b]], gbuf.at[b], gsem[b])

    def wait_gather(b):
        pltpu.make_async_copy(tab_hbm.at[pl.ds(0, BBLK)], gbuf.at[b], gsem[b]).wait()

    def transpose_unit(s, b):
        src = gbuf.at[b]
        dst = obuf.at[b]
        hv = []
        for g in range(BBLK // L):
            ids = ids_v[s, pl.ds(g * L, L)]
            hv.append(lax.shift_left(lax.bitwise_and(ids, 1), 6))

        @pl.loop(0, DIM, step=4)
        def _d(d0):
            for dd in range(4):
                d = d0 + dd
                for g in range(BBLK // L):
                    vals = plsc.load_gather(src, [iota + g * L, hv[g] + d])
                    dst[d, pl.ds(g * L, L)] = vals

    def fire_out(s, b):
        pltpu.async_copy(obuf.at[b], out_hbm.at[s, :, pl.ds(b0, BBLK)], osem[b])

    def wait_out(b):
        pltpu.make_async_copy(obuf.at[0], out_hbm.at[0, :, pl.ds(0, BBLK)], osem[b]).wait()

    prep(0, 0)
    fire_gather(0, 0)
    prep(1, 1)
    fire_gather(1, 1)

    @pl.loop(0, SEQ, step=2)
    def _step(c):
        for b in range(2):
            s = c + b
            wait_gather(b)

            @pl.when(s >= 2)
            def _():
                wait_out(b)

            transpose_unit(s, b)
            fire_out(s, b)

            @pl.when(s < SEQ - 2)
            def _fire_next():
                prep(s + 2, b)
                fire_gather(s + 2, b)

    wait_out(0)
    wait_out(1)


def kernel(input, table):
    tail_rm = lax.slice(table, (N_SLABS_FULL * 128, 0), (NUM_ROWS, DIM)).reshape(32, 128)
    tab_rm = _format_kernel(table.T, tail_rm)
    out3 = _gather_kernel(input.T, tab_rm)
    return out3.transpose(2, 0, 1)


# speedup vs baseline: 1.6319x; 1.6319x over previous
import functools

import jax
import jax.numpy as jnp
from jax import lax
from jax.experimental import pallas as pl
from jax.experimental.pallas import tpu as pltpu
from jax.experimental.pallas import tpu_sc as plsc

NUM_ROWS = 1000000
DIM = 64
BATCH = 4096
SEQ = 200

NC = 2
NS = 16
NW = NC * NS
L = 16
BBLK = BATCH // NW

TBLK = 1024
TGRID = (NUM_ROWS + TBLK - 1) // TBLK

_mesh = plsc.VectorSubcoreMesh(core_axis_name="c", subcore_axis_name="s")
_sc_params = pltpu.CompilerParams(use_tc_tiling_on_sc=True, needs_layout_passes=False)


def _tc_transpose_body(nt_ref, out_ref):
    out_ref[:, 0:DIM] = jnp.transpose(nt_ref[...], (1, 0))


_tc_transpose = pl.pallas_call(
    _tc_transpose_body,
    grid=(TGRID,),
    in_specs=[pl.BlockSpec((DIM, TBLK), lambda i: (0, i))],
    out_specs=pl.BlockSpec((TBLK, 128), lambda i: (i, 0)),
    out_shape=jax.ShapeDtypeStruct((NUM_ROWS, 128), jnp.float32),
)


@functools.partial(
    pl.kernel,
    out_type=jax.ShapeDtypeStruct((SEQ, DIM, BATCH), jnp.float32),
    mesh=_mesh,
    scratch_types=[
        pltpu.VMEM((SEQ, BBLK), jnp.int32),
        pltpu.VMEM((2, BBLK, 128), jnp.float32),
        pltpu.VMEM((2, DIM, BBLK), jnp.float32),
        pltpu.SemaphoreType.DMA,
        pltpu.SemaphoreType.DMA,
        pltpu.SemaphoreType.DMA,
        pltpu.SemaphoreType.DMA,
        pltpu.SemaphoreType.DMA,
    ],
    compiler_params=_sc_params,
)
def _gather_kernel(ids_hbm, tab_hbm, out_hbm, ids_v, gbuf, obuf, gs0, gs1, os0, os1, isem):
    w = lax.axis_index("s") * NC + lax.axis_index("c")
    b0 = pl.multiple_of(w * BBLK, BBLK)
    gsem = (gs0, gs1)
    osem = (os0, os1)
    rowv = [lax.broadcasted_iota(jnp.int32, (L,), 0) + g * L for g in range(BBLK // L)]

    pltpu.async_copy(ids_hbm.at[:, pl.ds(b0, BBLK)], ids_v, isem)
    pltpu.make_async_copy(ids_hbm.at[:, pl.ds(0, BBLK)], ids_v, isem).wait()

    def fire_gather(s, b):
        pltpu.async_copy(tab_hbm.at[ids_v.at[s]], gbuf.at[b], gsem[b])

    def wait_gather(b):
        pltpu.make_async_copy(tab_hbm.at[pl.ds(0, BBLK)], gbuf.at[b], gsem[b]).wait()

    def transpose_unit(b):
        src = gbuf.at[b]
        dst = obuf.at[b]

        @pl.loop(0, DIM, step=4)
        def _d(d0):
            vals = []
            for dd in range(4):
                col = jnp.broadcast_to(d0 + dd, (L,))
                for g in range(BBLK // L):
                    vals.append(plsc.load_gather(src, [rowv[g], col]))
            for dd in range(4):
                for g in range(BBLK // L):
                    dst[d0 + dd, pl.ds(g * L, L)] = vals[dd * (BBLK // L) + g]

    def fire_out(s, b):
        pltpu.async_copy(obuf.at[b], out_hbm.at[s, :, pl.ds(b0, BBLK)], osem[b])

    def wait_out(b):
        pltpu.make_async_copy(obuf.at[0], out_hbm.at[0, :, pl.ds(0, BBLK)], osem[b]).wait()

    fire_gather(0, 0)
    fire_gather(1, 1)

    @pl.loop(0, SEQ, step=2)
    def _step(c):
        for b in range(2):
            s = c + b
            wait_gather(b)

            @pl.when(s >= 2)
            def _():
                wait_out(b)

            transpose_unit(b)
            fire_out(s, b)

            @pl.when(s < SEQ - 2)
            def _fire_next():
                fire_gather(s + 2, b)

    wait_out(0)
    wait_out(1)


def kernel(input, table):
    tab_p = _tc_transpose(table.T)
    out3 = _gather_kernel(input.T, tab_p)
    return out3.transpose(2, 0, 1)
